# Initial kernel scaffold; baseline (speedup 1.0000x reference)
import jax
import jax.numpy as jnp
from jax.experimental import pallas as pl


def kernel(lst, gui):
    raise NotImplementedError("write your pallas kernel here")


# speedup vs baseline: 9.3832x; 9.3832x over previous
import jax
import jax.numpy as jnp
from jax.experimental import pallas as pl


def _body(mk_ref, ms_ref, lst_ref, gui_ref, ol_ref, og_ref):
    mk = mk_ref[0]
    ms = ms_ref[0]
    l = lst_ref[...]
    g = gui_ref[...]
    ol_ref[...] = l * mk + g * ms
    og_ref[...] = g * mk + l * ms


def kernel(lst, gui):
    N, C, H, W = lst.shape
    R = N * C
    lst3 = lst.reshape(R, H, W)
    gui3 = gui.reshape(R, H, W)

    spatial_mask = jax.random.randint(
        jax.random.key(42), (H,), 0, 2, dtype=jnp.int32)
    has0 = jnp.any(spatial_mask == 0)
    has1 = jnp.any(spatial_mask == 1)
    col = jnp.arange(W)
    m_keep = jnp.where(((col == W - 1) & has0) | ((col == W - 2) & has1),
                       1.0, 0.0).astype(lst.dtype)
    m_swap = jnp.where(((col == 0) & has0) | ((col == 1) & has1),
                       1.0, 0.0).astype(lst.dtype)
    m_keep = m_keep.reshape(1, 1, W)
    m_swap = m_swap.reshape(1, 1, W)

    B = 8
    grid = (R // B,)
    big_spec = pl.BlockSpec((B, H, W), lambda i: (i, 0, 0))
    vec_spec = pl.BlockSpec((1, 1, W), lambda i: (0, 0, 0))
    out_lst, out_gui = pl.pallas_call(
        _body,
        grid=grid,
        in_specs=[vec_spec, vec_spec, big_spec, big_spec],
        out_specs=[big_spec, big_spec],
        out_shape=[
            jax.ShapeDtypeStruct((R, H, W), lst.dtype),
            jax.ShapeDtypeStruct((R, H, W), gui.dtype),
        ],
    )(m_keep, m_swap, lst3, gui3)
    return (out_lst.reshape(N, C, H, W), out_gui.reshape(N, C, H, W))
